# baseline (speedup 1.0000x reference)
import functools

import jax
import jax.numpy as jnp
from jax import lax
from jax.experimental import pallas as pl
from jax.experimental.pallas import tpu as pltpu
from jax.experimental.pallas import tpu_sc as plsc

_L = 16
_NC, _NS = 2, 16
_NW = _NC * _NS
_UNROLL = 8
_NCHUNK = 8
_NBUF = 3


def kernel(logits, alt_counts, confidence, max_logit):
    n = logits.shape[0]
    k = confidence.shape[0]
    q = _UNROLL * _NCHUNK
    nv = -(-(n // _L) // _NW)
    nv = -(-nv // q) * q
    ch = nv * _L
    cnv = nv // _NCHUNK
    cch = cnv * _L
    kmax = k - 1

    pm = jnp.full((_L,), max_logit.astype(jnp.float32), jnp.float32)

    mesh = plsc.VectorSubcoreMesh(core_axis_name="c", subcore_axis_name="s")

    vmem = pltpu.VMEM
    sem = pltpu.SemaphoreType.DMA
    scratch = (
        [vmem((cch,), jnp.float32) for _ in range(_NBUF)]
        + [vmem((cch,), jnp.int32) for _ in range(_NBUF)]
        + [vmem((cch,), jnp.float32) for _ in range(_NBUF)]
        + [vmem((k,), jnp.float32), vmem((_L,), jnp.float32)]
        + [sem] * (3 * _NBUF + 2)
    )

    @functools.partial(
        pl.kernel,
        out_type=jax.ShapeDtypeStruct((n,), jnp.float32),
        mesh=mesh,
        compiler_params=pltpu.CompilerParams(needs_layout_passes=False),
        scratch_types=scratch,
    )
    def run(logits_hbm, counts_hbm, conf_hbm, pm_hbm, out_hbm, *bufs):
        lg_b = bufs[0:_NBUF]
        ct_b = bufs[_NBUF:2 * _NBUF]
        out_b = bufs[2 * _NBUF:3 * _NBUF]
        tab_v = bufs[3 * _NBUF]
        pm_v = bufs[3 * _NBUF + 1]
        slg = bufs[3 * _NBUF + 2:3 * _NBUF + 2 + _NBUF]
        sct = bufs[3 * _NBUF + 2 + _NBUF:3 * _NBUF + 2 + 2 * _NBUF]
        sout = bufs[3 * _NBUF + 2 + 2 * _NBUF:3 * _NBUF + 2 + 3 * _NBUF]
        stab = bufs[3 * _NBUF + 2 + 3 * _NBUF]
        spm = bufs[3 * _NBUF + 2 + 3 * _NBUF + 1]
        wid = lax.axis_index("s") * _NC + lax.axis_index("c")
        base = jnp.minimum(wid * ch, n - ch)

        def start_in(j):
            b = j % _NBUF
            hl = pltpu.async_copy(
                logits_hbm.at[pl.ds(base + j * cch, cch)], lg_b[b], slg[b])
            hc = pltpu.async_copy(
                counts_hbm.at[pl.ds(base + j * cch, cch)], ct_b[b], sct[b])
            return hl, hc

        hin = [None] * _NCHUNK
        hout = [None] * _NCHUNK
        hin[0] = start_in(0)
        htab = pltpu.async_copy(conf_hbm, tab_v, stab)
        hpm = pltpu.async_copy(pm_hbm, pm_v, spm)
        hin[1] = start_in(1)
        htab.wait()
        hpm.wait()
        pmv = pm_v[...]
        p2mv = pmv + pmv
        i2mv = 2.0 / pmv

        for j in range(_NCHUNK):
            if j + 2 < _NCHUNK:
                hin[j + 2] = start_in(j + 2)
            hin[j][0].wait()
            hin[j][1].wait()
            if j >= _NBUF:
                hout[j - _NBUF].wait()
            b = j % _NBUF
            lgb, ctb, outb = lg_b[b], ct_b[b], out_b[b]

            @plsc.parallel_loop(0, cnv, 1, unroll=_UNROLL)
            def body(i):
                x = lgb[pl.ds(i * _L, _L)]
                ci = jnp.minimum(ctb[pl.ds(i * _L, _L)], kmax)
                c = plsc.load_gather(tab_v, [ci])
                e = jnp.exp((x * c) * i2mv)
                outb[pl.ds(i * _L, _L)] = pmv - p2mv / (e + 1.0)

            hout[j] = pltpu.async_copy(
                outb, out_hbm.at[pl.ds(base + j * cch, cch)], sout[b])
        for j in range(max(0, _NCHUNK - _NBUF), _NCHUNK):
            hout[j].wait()

    return run(logits, alt_counts, confidence, pm)
